# baseline (speedup 1.0000x reference)
import jax
import jax.numpy as jnp
from jax import lax
from jax.experimental import pallas as pl
from jax.experimental.pallas import tpu as pltpu
from jax.experimental.pallas import tpu_sc as plsc

DEPTH = 4
N = 10000
E = 160000
H = 256
XW = 128
HW = H // 2

NC, NS = 2, 16
NW = NC * NS
GK = 128
GCH = 64
NP = 10240
ROWS_PT = NP // NS

EH = E // 2
EDGE_BLK = 1600
NODE_BLK = 2000

_sc_mesh = plsc.VectorSubcoreMesh(core_axis_name="c", subcore_axis_name="s")


def _silu(v):
    return v * jax.nn.sigmoid(v)


def _make_gather(eh):
    nchunks = eh // GCH
    gm = -(-nchunks // NW)
    if gm % 2 == 0:
        gm += 1

    def body(ptbl, qtbl, xtbl, srcv, dstv,
             epre, rout,
             idxs, idxd, bufp, bufq, bufxs, bufxd, sems):
        c = lax.axis_index("c")
        s = lax.axis_index("s")
        wid = s * NC + c

        def chunk_base(i):
            cid = jnp.minimum(wid + NW * i, nchunks - 1)
            return cid * GCH

        def issue(i, b):
            lbase = chunk_base(i)
            pltpu.sync_copy(srcv.at[pl.ds(lbase, GCH)], idxs.at[b])
            pltpu.sync_copy(dstv.at[pl.ds(lbase, GCH)], idxd.at[b])
            pltpu.async_copy(ptbl.at[idxs.at[b]], bufp.at[b], sems.at[b])
            pltpu.async_copy(qtbl.at[idxd.at[b]], bufq.at[b], sems.at[b])
            pltpu.async_copy(xtbl.at[idxs.at[b]], bufxs.at[b], sems.at[b])
            pltpu.async_copy(xtbl.at[idxd.at[b]], bufxd.at[b], sems.at[b])

        def drain(b):
            pltpu.make_async_copy(ptbl.at[idxs.at[b]], bufp.at[b],
                                  sems.at[b]).wait()
            pltpu.make_async_copy(qtbl.at[idxd.at[b]], bufq.at[b],
                                  sems.at[b]).wait()
            pltpu.make_async_copy(xtbl.at[idxs.at[b]], bufxs.at[b],
                                  sems.at[b]).wait()
            pltpu.make_async_copy(xtbl.at[idxd.at[b]], bufxd.at[b],
                                  sems.at[b]).wait()

        def compute_and_flush(i, b):
            lane = lax.iota(jnp.int32, 16)

            def edge(e, carry):
                xse = bufxs.at[b][e, pl.ds(0, 16)]
                xde = bufxd.at[b][e, pl.ds(0, 16)]
                r16 = xse - xde
                rr = r16 * r16
                d2 = rr[0] + rr[1] + rr[2]
                bufxs.at[b][e, pl.ds(0, 16)] = jnp.where(lane == 3, d2, r16)
                for j in range(H // 16):
                    sl = pl.ds(j * 16, 16)
                    bufp.at[b][e, sl] = bufp.at[b][e, sl] + bufq.at[b][e, sl]
                return carry

            lax.fori_loop(0, GCH, edge, 0)
            lbase = chunk_base(i)
            pltpu.sync_copy(bufp.at[b], epre.at[pl.ds(lbase, GCH)])
            pltpu.sync_copy(bufxs.at[b], rout.at[pl.ds(lbase, GCH)])

        issue(0, 0)

        def pair(j2, carry):
            a = 2 * j2
            issue(a + 1, 1)
            drain(0)
            compute_and_flush(a, 0)
            issue(a + 2, 0)
            drain(1)
            compute_and_flush(a + 1, 1)
            return carry

        lax.fori_loop(0, (gm - 1) // 2, pair, 0)
        drain(0)
        compute_and_flush(gm - 1, 0)

    return pl.kernel(
        body,
        out_type=[
            jax.ShapeDtypeStruct((eh, H), jnp.float32),
            jax.ShapeDtypeStruct((eh, XW), jnp.float32),
        ],
        mesh=_sc_mesh,
        scratch_types=[
            pltpu.VMEM((2, GCH), jnp.int32),
            pltpu.VMEM((2, GCH), jnp.int32),
            pltpu.VMEM((2, GCH, H), jnp.float32),
            pltpu.VMEM((2, GCH, H), jnp.float32),
            pltpu.VMEM((2, GCH, XW), jnp.float32),
            pltpu.VMEM((2, GCH, XW), jnp.float32),
            pltpu.SemaphoreType.DMA((2,)),
        ],
    )


_gather = _make_gather(EH)


def _make_scatter(eh):
    ept = eh // NS
    sfull, stail = ept // GK, ept % GK
    rfull, rtail = sfull, stail
    assert stail and rtail and stail % 8 == 0 and rtail % 8 == 0

    def body(m2, rc, dstv, zeros_h,
             agg2, xagg2,
             tbl, idxb, datb, idxt, datt, rcidxt, rcbt, sems):
        c = lax.axis_index("c")
        s = lax.axis_index("s")

        rows = pl.ds(s * ROWS_PT, ROWS_PT)
        pltpu.sync_copy(zeros_h.at[rows], tbl.at[rows])
        plsc.subcore_barrier()

        def issue(data, lbase, b):
            pltpu.async_copy(dstv.at[pl.ds(lbase, GK)], idxb.at[b],
                             sems.at[b])
            pltpu.async_copy(data.at[pl.ds(lbase, GK)], datb.at[b],
                             sems.at[b])

        def drain_add(data, lbase, b):
            pltpu.make_async_copy(dstv.at[pl.ds(lbase, GK)],
                                  idxb.at[b], sems.at[b]).wait()
            pltpu.make_async_copy(data.at[pl.ds(lbase, GK)], datb.at[b],
                                  sems.at[b]).wait()
            pltpu.sync_copy(datb.at[b], tbl.at[idxb.at[b]], add=True)

        def pipeline(data, base0, nfull):
            issue(data, base0, 0)

            def pair(j2, carry):
                a = 2 * j2
                issue(data, base0 + (a + 1) * GK, 1)
                drain_add(data, base0 + a * GK, 0)

                @pl.when(a + 2 < nfull)
                def _():
                    issue(data, base0 + (a + 2) * GK, 0)

                drain_add(data, base0 + (a + 1) * GK, 1)
                return carry

            lax.fori_loop(0, nfull // 2, pair, 0)
            if nfull % 2:
                drain_add(data, base0 + (nfull - 1) * GK, 0)

        mslab = m2.at[c]
        pipeline(mslab, s * ept, sfull)
        tbase = s * ept + sfull * GK
        pltpu.sync_copy(dstv.at[pl.ds(tbase, stail)], idxt)
        pltpu.sync_copy(mslab.at[pl.ds(tbase, stail)], datt)
        pltpu.sync_copy(datt, tbl.at[idxt], add=True)

        plsc.subcore_barrier()
        pltpu.sync_copy(tbl.at[rows], agg2.at[c, rows])
        pltpu.sync_copy(zeros_h.at[rows], tbl.at[rows])
        plsc.subcore_barrier()

        @pl.when(s % NC == c)
        def _():
            rbase0 = s * ept
            pipeline(rc, rbase0, rfull)
            rtbase = rbase0 + rfull * GK
            pltpu.sync_copy(dstv.at[pl.ds(rtbase, rtail)], rcidxt)
            pltpu.sync_copy(rc.at[pl.ds(rtbase, rtail)], rcbt)
            pltpu.sync_copy(rcbt, tbl.at[rcidxt], add=True)

        plsc.subcore_barrier()
        pltpu.sync_copy(tbl.at[rows], xagg2.at[c, rows])

    return pl.kernel(
        body,
        out_type=[
            jax.ShapeDtypeStruct((NC, NP, HW), jnp.float32),
            jax.ShapeDtypeStruct((NC, NP, XW), jnp.float32),
        ],
        mesh=_sc_mesh,
        scratch_types=[
            pltpu.VMEM_SHARED((NP, XW), jnp.float32),
            pltpu.VMEM((2, GK), jnp.int32),
            pltpu.VMEM((2, GK, XW), jnp.float32),
            pltpu.VMEM((stail,), jnp.int32),
            pltpu.VMEM((stail, XW), jnp.float32),
            pltpu.VMEM((rtail,), jnp.int32),
            pltpu.VMEM((rtail, XW), jnp.float32),
            pltpu.SemaphoreType.DMA((2,)),
        ],
    )


_scatter = _make_scatter(EH)


def _edge_kernel(epre_ref, r_ref, w512_ref, be2_ref, we2_ref, wx_ref, bx_ref,
                 m_ref, rc_ref):
    r = r_ref[...]
    d2 = r[:, 3:4]
    d2b = d2.astype(jnp.bfloat16).astype(jnp.float32)
    m1 = _silu(epre_ref[...] + d2b * w512_ref[...])
    mm = jnp.dot(m1, we2_ref[...], preferred_element_type=jnp.float32)
    m = _silu(mm + be2_ref[...])
    cp = jnp.dot(m, wx_ref[...], preferred_element_type=jnp.float32)
    coef = jnp.tanh(cp + bx_ref[...])
    m_ref[0] = m[:, :HW]
    m_ref[1] = m[:, HW:]
    lane = lax.broadcasted_iota(jnp.int32, (1, XW), 1)
    rc_ref[...] = jnp.where(lane == 3, 1.0, r * coef)


def _edge_stage(epre, r, w512b, be2, we2, wx, bx):
    eh = epre.shape[0]
    grid = (eh // EDGE_BLK,)
    be = lambda w: pl.BlockSpec((EDGE_BLK, w), lambda i: (i, 0))
    full = lambda a, b: pl.BlockSpec((a, b), lambda i: (0, 0))
    return pl.pallas_call(
        _edge_kernel,
        grid=grid,
        in_specs=[
            be(H), be(XW),
            full(1, H), full(1, H), full(H, H), full(H, 1), full(1, 1),
        ],
        out_specs=[
            pl.BlockSpec((NC, EDGE_BLK, HW), lambda i: (0, i, 0)),
            be(XW),
        ],
        out_shape=[
            jax.ShapeDtypeStruct((NC, eh, HW), jnp.float32),
            jax.ShapeDtypeStruct((eh, XW), jnp.float32),
        ],
    )(epre, r, w512b, be2, we2, wx, bx)


def _node0_kernel(h_ref, win_ref, bin_ref, wea_ref, web_ref, be1_ref,
                  h0_ref, pt_ref, qt_ref):
    h0 = jnp.dot(h_ref[...], win_ref[...],
                 preferred_element_type=jnp.float32) + bin_ref[...]
    h0_ref[...] = h0
    pt_ref[...] = jnp.dot(h0, wea_ref[...],
                          preferred_element_type=jnp.float32) + be1_ref[...]
    qt_ref[...] = jnp.dot(h0, web_ref[...], preferred_element_type=jnp.float32)


def _node0_stage(h, W_in, b_in, wea, web, be1):
    grid = (N // NODE_BLK,)
    bn = lambda w: pl.BlockSpec((NODE_BLK, w), lambda i: (i, 0))
    full = lambda a, b: pl.BlockSpec((a, b), lambda i: (0, 0))
    IN = h.shape[1]
    return pl.pallas_call(
        _node0_kernel,
        grid=grid,
        in_specs=[bn(IN), full(IN, H), full(1, H), full(H, H), full(H, H),
                  full(1, H)],
        out_specs=[bn(H), bn(H), bn(H)],
        out_shape=[
            jax.ShapeDtypeStruct((N, H), jnp.float32),
            jax.ShapeDtypeStruct((N, H), jnp.float32),
            jax.ShapeDtypeStruct((N, H), jnp.float32),
        ],
    )(h, W_in, b_in.reshape(1, H), wea, web, be1.reshape(1, H))


def _agg_sum(agg2a_ref, agg2b_ref, xagg2a_ref, xagg2b_ref):
    agg = jnp.concatenate([agg2a_ref[0] + agg2b_ref[0],
                           agg2a_ref[1] + agg2b_ref[1]], axis=-1)
    xagg = (xagg2a_ref[0] + xagg2a_ref[1]
            + xagg2b_ref[0] + xagg2b_ref[1])
    return agg, xagg


def _node_kernel(h_ref, agg2a_ref, agg2b_ref, xp_ref, xagg2a_ref, xagg2b_ref,
                 wn1a_ref, wn1b_ref, bn1_ref, wn2_ref, bn2_ref,
                 wea_ref, web_ref, be1n_ref,
                 hnew_ref, xpnew_ref, pt_ref, qt_ref):
    h = h_ref[...]
    agg, xagg = _agg_sum(agg2a_ref, agg2b_ref, xagg2a_ref, xagg2b_ref)
    hn = _silu(jnp.dot(h, wn1a_ref[...], preferred_element_type=jnp.float32)
               + jnp.dot(agg, wn1b_ref[...], preferred_element_type=jnp.float32)
               + bn1_ref[...])
    hn2 = jnp.dot(hn, wn2_ref[...],
                  preferred_element_type=jnp.float32) + bn2_ref[...]
    hN = _silu(hn2) + h
    hnew_ref[...] = hN
    deg = xagg[:, 3:4]
    lane = lax.broadcasted_iota(jnp.int32, (1, XW), 1)
    xpnew_ref[...] = xp_ref[...] + jnp.where(lane < 3, xagg / (deg + 1.0), 0.0)
    pt_ref[...] = jnp.dot(hN, wea_ref[...],
                          preferred_element_type=jnp.float32) + be1n_ref[...]
    qt_ref[...] = jnp.dot(hN, web_ref[...], preferred_element_type=jnp.float32)


def _node_stage(h, agg2a, agg2b, xp, xagg2a, xagg2b,
                wn1a, wn1b, bn1, wn2, bn2, wea, web, be1n):
    grid = (N // NODE_BLK,)
    bn = lambda w: pl.BlockSpec((NODE_BLK, w), lambda i: (i, 0))
    b2 = lambda w: pl.BlockSpec((NC, NODE_BLK, w), lambda i: (0, i, 0))
    full = lambda a, b: pl.BlockSpec((a, b), lambda i: (0, 0))
    return pl.pallas_call(
        _node_kernel,
        grid=grid,
        in_specs=[bn(H), b2(HW), b2(HW), bn(XW), b2(XW), b2(XW),
                  full(H, H), full(H, H), full(1, H), full(H, H), full(1, H),
                  full(H, H), full(H, H), full(1, H)],
        out_specs=[bn(H), bn(XW), bn(H), bn(H)],
        out_shape=[
            jax.ShapeDtypeStruct((N, H), jnp.float32),
            jax.ShapeDtypeStruct((N, XW), jnp.float32),
            jax.ShapeDtypeStruct((N, H), jnp.float32),
            jax.ShapeDtypeStruct((N, H), jnp.float32),
        ],
    )(h, agg2a, agg2b, xp, xagg2a, xagg2b, wn1a, wn1b, bn1.reshape(1, H),
      wn2, bn2.reshape(1, H), wea, web, be1n.reshape(1, H))


def _nodef_kernel(h_ref, agg2a_ref, agg2b_ref, xp_ref, xagg2a_ref,
                  xagg2b_ref,
                  wn1a_ref, wn1b_ref, bn1_ref, wn2_ref, bn2_ref,
                  wout_ref, bout_ref,
                  hout_ref, xpnew_ref):
    h = h_ref[...]
    agg, xagg = _agg_sum(agg2a_ref, agg2b_ref, xagg2a_ref, xagg2b_ref)
    hn = _silu(jnp.dot(h, wn1a_ref[...], preferred_element_type=jnp.float32)
               + jnp.dot(agg, wn1b_ref[...], preferred_element_type=jnp.float32)
               + bn1_ref[...])
    hn2 = jnp.dot(hn, wn2_ref[...],
                  preferred_element_type=jnp.float32) + bn2_ref[...]
    hN = _silu(hn2) + h
    hout_ref[...] = jnp.dot(hN, wout_ref[...],
                            preferred_element_type=jnp.float32) + bout_ref[...]
    deg = xagg[:, 3:4]
    lane = lax.broadcasted_iota(jnp.int32, (1, XW), 1)
    xpnew_ref[...] = xp_ref[...] + jnp.where(lane < 3, xagg / (deg + 1.0), 0.0)


def _nodef_stage(h, agg2a, agg2b, xp, xagg2a, xagg2b,
                 wn1a, wn1b, bn1, wn2, bn2, W_out, b_out):
    grid = (N // NODE_BLK,)
    OUT = W_out.shape[1]
    bn = lambda w: pl.BlockSpec((NODE_BLK, w), lambda i: (i, 0))
    b2 = lambda w: pl.BlockSpec((NC, NODE_BLK, w), lambda i: (0, i, 0))
    full = lambda a, b: pl.BlockSpec((a, b), lambda i: (0, 0))
    return pl.pallas_call(
        _nodef_kernel,
        grid=grid,
        in_specs=[bn(H), b2(HW), b2(HW), bn(XW), b2(XW), b2(XW),
                  full(H, H), full(H, H), full(1, H), full(H, H), full(1, H),
                  full(H, OUT), full(1, OUT)],
        out_specs=[bn(OUT), bn(XW)],
        out_shape=[
            jax.ShapeDtypeStruct((N, OUT), jnp.float32),
            jax.ShapeDtypeStruct((N, XW), jnp.float32),
        ],
    )(h, agg2a, agg2b, xp, xagg2a, xagg2b, wn1a, wn1b, bn1.reshape(1, H),
      wn2, bn2.reshape(1, H), W_out, b_out.reshape(1, OUT))


def kernel(h, x, edge_index, W_in, b_in, W_out, b_out, We1, be1, We2, be2,
           Wn1, bn1, Wn2, bn2, Wx, bx):
    srcv = edge_index[0]
    dstv = edge_index[1]
    srca, srcb = srcv[:EH], srcv[EH:]
    dsta, dstb = dstv[:EH], dstv[EH:]
    xp = jnp.concatenate([x, jnp.zeros((N, XW - 3), jnp.float32)], axis=1)
    zeros_h = jnp.zeros((NP, XW), jnp.float32)

    h, P, Q = _node0_stage(h, W_in, b_in, We1[0, :H], We1[0, H:2 * H], be1[0])

    for l in range(DEPTH):
        w512b = (We1[l, 2 * H].reshape(1, H)
                 .astype(jnp.bfloat16).astype(jnp.float32))
        be2l = be2[l].reshape(1, H)
        bxl = bx[l].reshape(1, 1)

        eprea, ra = _gather(P, Q, xp, srca, dsta)
        m2a, rca = _edge_stage(eprea, ra, w512b, be2l, We2[l], Wx[l], bxl)
        epreb, rb = _gather(P, Q, xp, srcb, dstb)
        agg2a, xagg2a = _scatter(m2a, rca, dsta, zeros_h)
        m2b, rcb = _edge_stage(epreb, rb, w512b, be2l, We2[l], Wx[l], bxl)
        agg2b, xagg2b = _scatter(m2b, rcb, dstb, zeros_h)

        if l < DEPTH - 1:
            h, xp, P, Q = _node_stage(
                h, agg2a, agg2b, xp, xagg2a, xagg2b,
                Wn1[l, :H], Wn1[l, H:], bn1[l], Wn2[l], bn2[l],
                We1[l + 1, :H], We1[l + 1, H:2 * H], be1[l + 1])
        else:
            h, xp = _nodef_stage(
                h, agg2a, agg2b, xp, xagg2a, xagg2b,
                Wn1[l, :H], Wn1[l, H:], bn1[l], Wn2[l], bn2[l],
                W_out, b_out)
    return (h, xp[:, :3])


# speedup vs baseline: 1.0143x; 1.0143x over previous
import functools

import jax
import jax.numpy as jnp
from jax import lax
from jax.experimental import pallas as pl
from jax.experimental.pallas import tpu as pltpu
from jax.experimental.pallas import tpu_sc as plsc

DEPTH = 4
N = 10000
E = 160000
H = 256
XW = 128
HW = H // 2

NC, NS = 2, 16
NW = NC * NS
EPW = E // NW
EPT = E // NS
GK = 128
NGC = EPW // GK
NP = 10240
ROWS_PT = NP // NS

SFULL = EPT // GK
STAIL = EPT - SFULL * GK
RFULL = EPW // GK
RTAIL = EPW - RFULL * GK

EDGE_BLK = 2000
NODE_BLK = 2000

_sc_mesh = plsc.VectorSubcoreMesh(core_axis_name="c", subcore_axis_name="s")


def _silu(v):
    return v * jax.nn.sigmoid(v)


GCH = 64
GM = (EPW + GCH - 1) // GCH


def _gather_chunk_base(wid, i):
    return wid * EPW + jnp.minimum(i * GCH, EPW - GCH)


def _gather_body(ptbl, qtbl, xtbl, srcv, dstv,
                 epre, rout,
                 idxs, idxd, bufp, bufq, bufxs, bufxd, sems):
    c = lax.axis_index("c")
    s = lax.axis_index("s")
    wid = s * NC + c

    def issue(i, b):
        base = _gather_chunk_base(wid, i)
        pltpu.sync_copy(srcv.at[pl.ds(base, GCH)], idxs.at[b])
        pltpu.sync_copy(dstv.at[pl.ds(base, GCH)], idxd.at[b])
        pltpu.async_copy(ptbl.at[idxs.at[b]], bufp.at[b], sems.at[b])
        pltpu.async_copy(qtbl.at[idxd.at[b]], bufq.at[b], sems.at[b])
        pltpu.async_copy(xtbl.at[idxs.at[b]], bufxs.at[b], sems.at[b])
        pltpu.async_copy(xtbl.at[idxd.at[b]], bufxd.at[b], sems.at[b])

    def drain(b):
        pltpu.make_async_copy(ptbl.at[idxs.at[b]], bufp.at[b], sems.at[b]).wait()
        pltpu.make_async_copy(qtbl.at[idxd.at[b]], bufq.at[b], sems.at[b]).wait()
        pltpu.make_async_copy(xtbl.at[idxs.at[b]], bufxs.at[b], sems.at[b]).wait()
        pltpu.make_async_copy(xtbl.at[idxd.at[b]], bufxd.at[b], sems.at[b]).wait()

    def compute_and_flush(i, b):
        lane = lax.iota(jnp.int32, 16)

        def edge(e, carry):
            xse = bufxs.at[b][e, pl.ds(0, 16)]
            xde = bufxd.at[b][e, pl.ds(0, 16)]
            r16 = xse - xde
            rr = r16 * r16
            d2 = rr[0] + rr[1] + rr[2]
            bufxs.at[b][e, pl.ds(0, 16)] = jnp.where(lane == 3, d2, r16)
            for j in range(H // 16):
                sl = pl.ds(j * 16, 16)
                ep = bufp.at[b][e, sl] + bufq.at[b][e, sl]
                bufp.at[b][e, sl] = ep
            return carry

        lax.fori_loop(0, GCH, edge, 0)
        base = _gather_chunk_base(wid, i)
        pltpu.sync_copy(bufp.at[b], epre.at[pl.ds(base, GCH)])
        pltpu.sync_copy(bufxs.at[b], rout.at[pl.ds(base, GCH)])

    issue(0, 0)

    def pair(j2, carry):
        a = 2 * j2
        issue(a + 1, 1)
        drain(0)
        compute_and_flush(a, 0)
        issue(a + 2, 0)
        drain(1)
        compute_and_flush(a + 1, 1)
        return carry

    lax.fori_loop(0, (GM - 1) // 2, pair, 0)
    drain(0)
    compute_and_flush(GM - 1, 0)


_gather = pl.kernel(
    _gather_body,
    out_type=[
        jax.ShapeDtypeStruct((E, H), jnp.float32),
        jax.ShapeDtypeStruct((E, XW), jnp.float32),
    ],
    mesh=_sc_mesh,
    scratch_types=[
        pltpu.VMEM((2, GCH), jnp.int32),
        pltpu.VMEM((2, GCH), jnp.int32),
        pltpu.VMEM((2, GCH, H), jnp.float32),
        pltpu.VMEM((2, GCH, H), jnp.float32),
        pltpu.VMEM((2, GCH, XW), jnp.float32),
        pltpu.VMEM((2, GCH, XW), jnp.float32),
        pltpu.SemaphoreType.DMA((2,)),
    ],
)


def _scatter_body(m2, rc, dstv, zeros_h,
                  agg2, xagg2,
                  tbl, idxb, datb, idxt, datt, rcidxt, rcbt, sems):
    c = lax.axis_index("c")
    s = lax.axis_index("s")

    rows = pl.ds(s * ROWS_PT, ROWS_PT)
    pltpu.sync_copy(zeros_h.at[rows], tbl.at[rows])
    plsc.subcore_barrier()

    def issue(data, base, b):
        pltpu.async_copy(dstv.at[pl.ds(base, GK)], idxb.at[b], sems.at[b])
        pltpu.async_copy(data.at[pl.ds(base, GK)], datb.at[b], sems.at[b])

    def drain_add(data, base, b):
        pltpu.make_async_copy(dstv.at[pl.ds(base, GK)], idxb.at[b],
                              sems.at[b]).wait()
        pltpu.make_async_copy(data.at[pl.ds(base, GK)], datb.at[b],
                              sems.at[b]).wait()
        pltpu.sync_copy(datb.at[b], tbl.at[idxb.at[b]], add=True)

    def pipeline(data, base0, nfull):
        issue(data, base0, 0)

        def pair(j2, carry):
            a = 2 * j2
            issue(data, base0 + (a + 1) * GK, 1)
            drain_add(data, base0 + a * GK, 0)

            @pl.when(a + 2 < nfull)
            def _():
                issue(data, base0 + (a + 2) * GK, 0)

            drain_add(data, base0 + (a + 1) * GK, 1)
            return carry

        lax.fori_loop(0, nfull // 2, pair, 0)
        if nfull % 2:
            drain_add(data, base0 + (nfull - 1) * GK, 0)

    mslab = m2.at[c]
    pipeline(mslab, s * EPT, SFULL)
    tbase = s * EPT + SFULL * GK
    pltpu.sync_copy(dstv.at[pl.ds(tbase, STAIL)], idxt)
    pltpu.sync_copy(mslab.at[pl.ds(tbase, STAIL)], datt)
    pltpu.sync_copy(datt, tbl.at[idxt], add=True)

    plsc.subcore_barrier()
    pltpu.sync_copy(tbl.at[rows], agg2.at[c, rows])
    pltpu.sync_copy(zeros_h.at[rows], tbl.at[rows])
    plsc.subcore_barrier()

    rbase0 = (s * NC + c) * EPW
    pipeline(rc, rbase0, RFULL)
    rtbase = rbase0 + RFULL * GK
    pltpu.sync_copy(dstv.at[pl.ds(rtbase, RTAIL)], rcidxt)
    pltpu.sync_copy(rc.at[pl.ds(rtbase, RTAIL)], rcbt)
    pltpu.sync_copy(rcbt, tbl.at[rcidxt], add=True)

    plsc.subcore_barrier()
    pltpu.sync_copy(tbl.at[rows], xagg2.at[c, rows])


_scatter = pl.kernel(
    _scatter_body,
    out_type=[
        jax.ShapeDtypeStruct((NC, NP, HW), jnp.float32),
        jax.ShapeDtypeStruct((NC, NP, XW), jnp.float32),
    ],
    mesh=_sc_mesh,
    scratch_types=[
        pltpu.VMEM_SHARED((NP, XW), jnp.float32),
        pltpu.VMEM((2, GK), jnp.int32),
        pltpu.VMEM((2, GK, XW), jnp.float32),
        pltpu.VMEM((STAIL,), jnp.int32),
        pltpu.VMEM((STAIL, XW), jnp.float32),
        pltpu.VMEM((RTAIL,), jnp.int32),
        pltpu.VMEM((RTAIL, XW), jnp.float32),
        pltpu.SemaphoreType.DMA((2,)),
    ],
)


def _edge_kernel(epre_ref, r_ref, w512_ref, be2_ref, we2_ref, wx_ref, bx_ref,
                 m_ref, rc_ref):
    r = r_ref[...]
    d2 = r[:, 3:4]
    d2b = d2.astype(jnp.bfloat16).astype(jnp.float32)
    m1 = _silu(epre_ref[...] + d2b * w512_ref[...])
    mm = jnp.dot(m1, we2_ref[...], preferred_element_type=jnp.float32)
    m = _silu(mm + be2_ref[...])
    cp = jnp.dot(m, wx_ref[...], preferred_element_type=jnp.float32)
    coef = jnp.tanh(cp + bx_ref[...])
    m_ref[0] = m[:, :HW]
    m_ref[1] = m[:, HW:]
    lane = lax.broadcasted_iota(jnp.int32, (1, XW), 1)
    rc_ref[...] = jnp.where(lane == 3, 1.0, r * coef)


def _edge_stage(epre, r, w512b, be2, we2, wx, bx):
    grid = (E // EDGE_BLK,)
    be = lambda w: pl.BlockSpec((EDGE_BLK, w), lambda i: (i, 0))
    full = lambda a, b: pl.BlockSpec((a, b), lambda i: (0, 0))
    return pl.pallas_call(
        _edge_kernel,
        grid=grid,
        in_specs=[
            be(H), be(XW),
            full(1, H), full(1, H), full(H, H), full(H, 1), full(1, 1),
        ],
        out_specs=[
            pl.BlockSpec((NC, EDGE_BLK, HW), lambda i: (0, i, 0)),
            be(XW),
        ],
        out_shape=[
            jax.ShapeDtypeStruct((NC, E, HW), jnp.float32),
            jax.ShapeDtypeStruct((E, XW), jnp.float32),
        ],
    )(epre, r, w512b, be2, we2, wx, bx)


def _node0_kernel(h_ref, win_ref, bin_ref, wea_ref, web_ref, be1_ref,
                  h0_ref, pt_ref, qt_ref):
    h0 = jnp.dot(h_ref[...], win_ref[...],
                 preferred_element_type=jnp.float32) + bin_ref[...]
    h0_ref[...] = h0
    pt_ref[...] = jnp.dot(h0, wea_ref[...],
                          preferred_element_type=jnp.float32) + be1_ref[...]
    qt_ref[...] = jnp.dot(h0, web_ref[...], preferred_element_type=jnp.float32)


def _node0_stage(h, W_in, b_in, wea, web, be1):
    grid = (N // NODE_BLK,)
    bn = lambda w: pl.BlockSpec((NODE_BLK, w), lambda i: (i, 0))
    full = lambda a, b: pl.BlockSpec((a, b), lambda i: (0, 0))
    IN = h.shape[1]
    return pl.pallas_call(
        _node0_kernel,
        grid=grid,
        in_specs=[bn(IN), full(IN, H), full(1, H), full(H, H), full(H, H),
                  full(1, H)],
        out_specs=[bn(H), bn(H), bn(H)],
        out_shape=[
            jax.ShapeDtypeStruct((N, H), jnp.float32),
            jax.ShapeDtypeStruct((N, H), jnp.float32),
            jax.ShapeDtypeStruct((N, H), jnp.float32),
        ],
    )(h, W_in, b_in.reshape(1, H), wea, web, be1.reshape(1, H))


def _node_kernel(h_ref, agg2_ref, xp_ref, xagg2_ref,
                 wn1a_ref, wn1b_ref, bn1_ref, wn2_ref, bn2_ref,
                 wea_ref, web_ref, be1n_ref,
                 hnew_ref, xpnew_ref, pt_ref, qt_ref):
    h = h_ref[...]
    agg = jnp.concatenate([agg2_ref[0], agg2_ref[1]], axis=-1)
    hn = _silu(jnp.dot(h, wn1a_ref[...], preferred_element_type=jnp.float32)
               + jnp.dot(agg, wn1b_ref[...], preferred_element_type=jnp.float32)
               + bn1_ref[...])
    hn2 = jnp.dot(hn, wn2_ref[...],
                  preferred_element_type=jnp.float32) + bn2_ref[...]
    hN = _silu(hn2) + h
    hnew_ref[...] = hN
    xagg = xagg2_ref[0] + xagg2_ref[1]
    deg = xagg[:, 3:4]
    lane = lax.broadcasted_iota(jnp.int32, (1, XW), 1)
    xpnew_ref[...] = xp_ref[...] + jnp.where(lane < 3, xagg / (deg + 1.0), 0.0)
    pt_ref[...] = jnp.dot(hN, wea_ref[...],
                          preferred_element_type=jnp.float32) + be1n_ref[...]
    qt_ref[...] = jnp.dot(hN, web_ref[...], preferred_element_type=jnp.float32)


def _node_stage(h, agg2, xp, xagg2, wn1a, wn1b, bn1, wn2, bn2, wea, web, be1n):
    grid = (N // NODE_BLK,)
    bn = lambda w: pl.BlockSpec((NODE_BLK, w), lambda i: (i, 0))
    b2 = lambda w: pl.BlockSpec((NC, NODE_BLK, w), lambda i: (0, i, 0))
    full = lambda a, b: pl.BlockSpec((a, b), lambda i: (0, 0))
    return pl.pallas_call(
        _node_kernel,
        grid=grid,
        in_specs=[bn(H), b2(HW), bn(XW), b2(XW),
                  full(H, H), full(H, H), full(1, H), full(H, H), full(1, H),
                  full(H, H), full(H, H), full(1, H)],
        out_specs=[bn(H), bn(XW), bn(H), bn(H)],
        out_shape=[
            jax.ShapeDtypeStruct((N, H), jnp.float32),
            jax.ShapeDtypeStruct((N, XW), jnp.float32),
            jax.ShapeDtypeStruct((N, H), jnp.float32),
            jax.ShapeDtypeStruct((N, H), jnp.float32),
        ],
    )(h, agg2, xp, xagg2, wn1a, wn1b, bn1.reshape(1, H), wn2,
      bn2.reshape(1, H), wea, web, be1n.reshape(1, H))


def _nodef_kernel(h_ref, agg2_ref, xp_ref, xagg2_ref,
                  wn1a_ref, wn1b_ref, bn1_ref, wn2_ref, bn2_ref,
                  wout_ref, bout_ref,
                  hout_ref, xpnew_ref):
    h = h_ref[...]
    agg = jnp.concatenate([agg2_ref[0], agg2_ref[1]], axis=-1)
    hn = _silu(jnp.dot(h, wn1a_ref[...], preferred_element_type=jnp.float32)
               + jnp.dot(agg, wn1b_ref[...], preferred_element_type=jnp.float32)
               + bn1_ref[...])
    hn2 = jnp.dot(hn, wn2_ref[...],
                  preferred_element_type=jnp.float32) + bn2_ref[...]
    hN = _silu(hn2) + h
    hout_ref[...] = jnp.dot(hN, wout_ref[...],
                            preferred_element_type=jnp.float32) + bout_ref[...]
    xagg = xagg2_ref[0] + xagg2_ref[1]
    deg = xagg[:, 3:4]
    lane = lax.broadcasted_iota(jnp.int32, (1, XW), 1)
    xpnew_ref[...] = xp_ref[...] + jnp.where(lane < 3, xagg / (deg + 1.0), 0.0)


def _nodef_stage(h, agg2, xp, xagg2, wn1a, wn1b, bn1, wn2, bn2, W_out, b_out):
    grid = (N // NODE_BLK,)
    OUT = W_out.shape[1]
    bn = lambda w: pl.BlockSpec((NODE_BLK, w), lambda i: (i, 0))
    b2 = lambda w: pl.BlockSpec((NC, NODE_BLK, w), lambda i: (0, i, 0))
    full = lambda a, b: pl.BlockSpec((a, b), lambda i: (0, 0))
    return pl.pallas_call(
        _nodef_kernel,
        grid=grid,
        in_specs=[bn(H), b2(HW), bn(XW), b2(XW),
                  full(H, H), full(H, H), full(1, H), full(H, H), full(1, H),
                  full(H, OUT), full(1, OUT)],
        out_specs=[bn(OUT), bn(XW)],
        out_shape=[
            jax.ShapeDtypeStruct((N, OUT), jnp.float32),
            jax.ShapeDtypeStruct((N, XW), jnp.float32),
        ],
    )(h, agg2, xp, xagg2, wn1a, wn1b, bn1.reshape(1, H), wn2,
      bn2.reshape(1, H), W_out, b_out.reshape(1, OUT))


def kernel(h, x, edge_index, W_in, b_in, W_out, b_out, We1, be1, We2, be2,
           Wn1, bn1, Wn2, bn2, Wx, bx):
    srcv = edge_index[0]
    dstv = edge_index[1]
    xp = jnp.concatenate([x, jnp.zeros((N, XW - 3), jnp.float32)], axis=1)
    zeros_h = jnp.zeros((NP, XW), jnp.float32)

    h, P, Q = _node0_stage(h, W_in, b_in, We1[0, :H], We1[0, H:2 * H], be1[0])

    for l in range(DEPTH):
        w512b = We1[l, 2 * H].reshape(1, H).astype(jnp.bfloat16).astype(jnp.float32)
        epre, r = _gather(P, Q, xp, srcv, dstv)
        m2, rc = _edge_stage(epre, r, w512b,
                             be2[l].reshape(1, H),
                             We2[l],
                             Wx[l], bx[l].reshape(1, 1))
        agg2, xagg2 = _scatter(m2, rc, dstv, zeros_h)
        if l < DEPTH - 1:
            h, xp, P, Q = _node_stage(
                h, agg2, xp, xagg2,
                Wn1[l, :H], Wn1[l, H:], bn1[l], Wn2[l], bn2[l],
                We1[l + 1, :H], We1[l + 1, H:2 * H], be1[l + 1])
        else:
            h, xp = _nodef_stage(
                h, agg2, xp, xagg2,
                Wn1[l, :H], Wn1[l, H:], bn1[l], Wn2[l], bn2[l],
                W_out, b_out)
    return (h, xp[:, :3])


# speedup vs baseline: 1.0403x; 1.0256x over previous
import jax
import jax.numpy as jnp
from jax import lax
from jax.experimental import pallas as pl
from jax.experimental.pallas import tpu as pltpu
from jax.experimental.pallas import tpu_sc as plsc

DEPTH = 4
N = 10000
E = 160000
H = 256
XW = 128
HW = H // 2

NC, NS = 2, 16
NW = NC * NS
EPW = E // NW
EPT = E // NS
GK = 128
NP = 10240
ROWS_PT = NP // NS

SFULL = EPT // GK
STAIL = EPT - SFULL * GK
RFULL = EPW // GK
RTAIL = EPW - RFULL * GK

EDGE_BLK = 2000
NODE_BLK = 2000

_sc_mesh = plsc.VectorSubcoreMesh(core_axis_name="c", subcore_axis_name="s")


def _silu(v):
    return v * jax.nn.sigmoid(v)


GCH = 80
GM = (EPW + GCH - 1) // GCH


def _gather_chunk_base(wid, i):
    return wid * EPW + jnp.minimum(i * GCH, EPW - GCH)


def _gather_body(ptbl, qtbl, xtbl, srcv, dstv,
                 epre, rout,
                 idxs, idxd, bufp, bufq, bufxs, bufxd, sems):
    c = lax.axis_index("c")
    s = lax.axis_index("s")
    wid = s * NC + c

    def issue(i, b):
        base = _gather_chunk_base(wid, i)
        pltpu.sync_copy(srcv.at[pl.ds(base, GCH)], idxs.at[b])
        pltpu.sync_copy(dstv.at[pl.ds(base, GCH)], idxd.at[b])
        pltpu.async_copy(ptbl.at[idxs.at[b]], bufp.at[b], sems.at[b])
        pltpu.async_copy(qtbl.at[idxd.at[b]], bufq.at[b], sems.at[b])
        pltpu.async_copy(xtbl.at[idxs.at[b]], bufxs.at[b], sems.at[b])
        pltpu.async_copy(xtbl.at[idxd.at[b]], bufxd.at[b], sems.at[b])

    def drain(b):
        pltpu.make_async_copy(ptbl.at[idxs.at[b]], bufp.at[b], sems.at[b]).wait()
        pltpu.make_async_copy(qtbl.at[idxd.at[b]], bufq.at[b], sems.at[b]).wait()
        pltpu.make_async_copy(xtbl.at[idxs.at[b]], bufxs.at[b], sems.at[b]).wait()
        pltpu.make_async_copy(xtbl.at[idxd.at[b]], bufxd.at[b], sems.at[b]).wait()

    def compute_and_flush(i, b):
        lane = lax.iota(jnp.int32, 16)

        def edge(e, carry):
            xse = bufxs.at[b][e, pl.ds(0, 16)]
            xde = bufxd.at[b][e, pl.ds(0, 16)]
            r16 = xse - xde
            rr = r16 * r16
            d2 = rr[0] + rr[1] + rr[2]
            bufxs.at[b][e, pl.ds(0, 16)] = jnp.where(lane == 3, d2, r16)
            for j in range(H // 16):
                sl = pl.ds(j * 16, 16)
                ep = bufp.at[b][e, sl] + bufq.at[b][e, sl]
                bufp.at[b][e, sl] = ep
            return carry

        lax.fori_loop(0, GCH, edge, 0)
        base = _gather_chunk_base(wid, i)
        pltpu.sync_copy(bufp.at[b], epre.at[pl.ds(base, GCH)])
        pltpu.sync_copy(bufxs.at[b], rout.at[pl.ds(base, GCH)])

    issue(0, 0)

    def pair(j2, carry):
        a = 2 * j2
        issue(a + 1, 1)
        drain(0)
        compute_and_flush(a, 0)
        issue(a + 2, 0)
        drain(1)
        compute_and_flush(a + 1, 1)
        return carry

    lax.fori_loop(0, (GM - 1) // 2, pair, 0)
    drain(0)
    compute_and_flush(GM - 1, 0)


_gather = pl.kernel(
    _gather_body,
    out_type=[
        jax.ShapeDtypeStruct((E, H), jnp.float32),
        jax.ShapeDtypeStruct((E, XW), jnp.float32),
    ],
    mesh=_sc_mesh,
    scratch_types=[
        pltpu.VMEM((2, GCH), jnp.int32),
        pltpu.VMEM((2, GCH), jnp.int32),
        pltpu.VMEM((2, GCH, H), jnp.float32),
        pltpu.VMEM((2, GCH, H), jnp.float32),
        pltpu.VMEM((2, GCH, XW), jnp.float32),
        pltpu.VMEM((2, GCH, XW), jnp.float32),
        pltpu.SemaphoreType.DMA((2,)),
    ],
)


def _scatter_body(m2, rc, dstv, zeros_h,
                  agg2, xagg2,
                  tbl, idxb, datb, idxt, datt, rcidxt, rcbt, sems):
    c = lax.axis_index("c")
    s = lax.axis_index("s")

    rows = pl.ds(s * ROWS_PT, ROWS_PT)
    pltpu.sync_copy(zeros_h.at[rows], tbl.at[rows])
    plsc.subcore_barrier()

    def issue(data, base, b):
        pltpu.async_copy(dstv.at[pl.ds(base, GK)], idxb.at[b], sems.at[b])
        pltpu.async_copy(data.at[pl.ds(base, GK)], datb.at[b], sems.at[b])

    def drain_add(data, base, b):
        pltpu.make_async_copy(dstv.at[pl.ds(base, GK)], idxb.at[b],
                              sems.at[b]).wait()
        pltpu.make_async_copy(data.at[pl.ds(base, GK)], datb.at[b],
                              sems.at[b]).wait()
        pltpu.sync_copy(datb.at[b], tbl.at[idxb.at[b]], add=True)

    def pipeline(data, base0, nfull):
        issue(data, base0, 0)

        def pair(j2, carry):
            a = 2 * j2
            issue(data, base0 + (a + 1) * GK, 1)
            drain_add(data, base0 + a * GK, 0)

            @pl.when(a + 2 < nfull)
            def _():
                issue(data, base0 + (a + 2) * GK, 0)

            drain_add(data, base0 + (a + 1) * GK, 1)
            return carry

        lax.fori_loop(0, nfull // 2, pair, 0)
        if nfull % 2:
            drain_add(data, base0 + (nfull - 1) * GK, 0)

    mslab = m2.at[c]
    pipeline(mslab, s * EPT, SFULL)
    tbase = s * EPT + SFULL * GK
    pltpu.sync_copy(dstv.at[pl.ds(tbase, STAIL)], idxt)
    pltpu.sync_copy(mslab.at[pl.ds(tbase, STAIL)], datt)
    pltpu.sync_copy(datt, tbl.at[idxt], add=True)

    plsc.subcore_barrier()
    pltpu.sync_copy(tbl.at[rows], agg2.at[c, rows])
    pltpu.sync_copy(zeros_h.at[rows], tbl.at[rows])
    plsc.subcore_barrier()

    rbase0 = (s * NC + c) * EPW
    pipeline(rc, rbase0, RFULL)
    rtbase = rbase0 + RFULL * GK
    pltpu.sync_copy(dstv.at[pl.ds(rtbase, RTAIL)], rcidxt)
    pltpu.sync_copy(rc.at[pl.ds(rtbase, RTAIL)], rcbt)
    pltpu.sync_copy(rcbt, tbl.at[rcidxt], add=True)

    plsc.subcore_barrier()
    pltpu.sync_copy(tbl.at[rows], xagg2.at[c, rows])


_scatter = pl.kernel(
    _scatter_body,
    out_type=[
        jax.ShapeDtypeStruct((NC, NP, HW), jnp.float32),
        jax.ShapeDtypeStruct((NC, NP, XW), jnp.float32),
    ],
    mesh=_sc_mesh,
    scratch_types=[
        pltpu.VMEM_SHARED((NP, XW), jnp.float32),
        pltpu.VMEM((2, GK), jnp.int32),
        pltpu.VMEM((2, GK, XW), jnp.float32),
        pltpu.VMEM((STAIL,), jnp.int32),
        pltpu.VMEM((STAIL, XW), jnp.float32),
        pltpu.VMEM((RTAIL,), jnp.int32),
        pltpu.VMEM((RTAIL, XW), jnp.float32),
        pltpu.SemaphoreType.DMA((2,)),
    ],
)


def _edge_kernel(epre_ref, r_ref, w512_ref, be2_ref, we2_ref, wx_ref, bx_ref,
                 m_ref, rc_ref):
    r = r_ref[...]
    d2 = r[:, 3:4]
    d2b = d2.astype(jnp.bfloat16).astype(jnp.float32)
    m1 = _silu(epre_ref[...] + d2b * w512_ref[...])
    mm = jnp.dot(m1, we2_ref[...], preferred_element_type=jnp.float32)
    m = _silu(mm + be2_ref[...])
    cp = jnp.dot(m, wx_ref[...], preferred_element_type=jnp.float32)
    coef = jnp.tanh(cp + bx_ref[...])
    m_ref[0] = m[:, :HW]
    m_ref[1] = m[:, HW:]
    lane = lax.broadcasted_iota(jnp.int32, (1, XW), 1)
    rc_ref[...] = jnp.where(lane == 3, 1.0, r * coef)


def _edge_stage(epre, r, w512b, be2, we2, wx, bx):
    grid = (E // EDGE_BLK,)
    be = lambda w: pl.BlockSpec((EDGE_BLK, w), lambda i: (i, 0))
    full = lambda a, b: pl.BlockSpec((a, b), lambda i: (0, 0))
    return pl.pallas_call(
        _edge_kernel,
        grid=grid,
        in_specs=[
            be(H), be(XW),
            full(1, H), full(1, H), full(H, H), full(H, 1), full(1, 1),
        ],
        out_specs=[
            pl.BlockSpec((NC, EDGE_BLK, HW), lambda i: (0, i, 0)),
            be(XW),
        ],
        out_shape=[
            jax.ShapeDtypeStruct((NC, E, HW), jnp.float32),
            jax.ShapeDtypeStruct((E, XW), jnp.float32),
        ],
    )(epre, r, w512b, be2, we2, wx, bx)


def _node0_kernel(h_ref, win_ref, bin_ref, wea_ref, web_ref, be1_ref,
                  h0_ref, pt_ref, qt_ref):
    h0 = jnp.dot(h_ref[...], win_ref[...],
                 preferred_element_type=jnp.float32) + bin_ref[...]
    h0_ref[...] = h0
    pt_ref[...] = jnp.dot(h0, wea_ref[...],
                          preferred_element_type=jnp.float32) + be1_ref[...]
    qt_ref[...] = jnp.dot(h0, web_ref[...], preferred_element_type=jnp.float32)


def _node0_stage(h, W_in, b_in, wea, web, be1):
    grid = (N // NODE_BLK,)
    bn = lambda w: pl.BlockSpec((NODE_BLK, w), lambda i: (i, 0))
    full = lambda a, b: pl.BlockSpec((a, b), lambda i: (0, 0))
    IN = h.shape[1]
    return pl.pallas_call(
        _node0_kernel,
        grid=grid,
        in_specs=[bn(IN), full(IN, H), full(1, H), full(H, H), full(H, H),
                  full(1, H)],
        out_specs=[bn(H), bn(H), bn(H)],
        out_shape=[
            jax.ShapeDtypeStruct((N, H), jnp.float32),
            jax.ShapeDtypeStruct((N, H), jnp.float32),
            jax.ShapeDtypeStruct((N, H), jnp.float32),
        ],
    )(h, W_in, b_in.reshape(1, H), wea, web, be1.reshape(1, H))


def _node_kernel(h_ref, agg2_ref, xp_ref, xagg2_ref,
                 wn1a_ref, wn1b_ref, bn1_ref, wn2_ref, bn2_ref,
                 wea_ref, web_ref, be1n_ref,
                 hnew_ref, xpnew_ref, pt_ref, qt_ref):
    h = h_ref[...]
    agg = jnp.concatenate([agg2_ref[0], agg2_ref[1]], axis=-1)
    hn = _silu(jnp.dot(h, wn1a_ref[...], preferred_element_type=jnp.float32)
               + jnp.dot(agg, wn1b_ref[...], preferred_element_type=jnp.float32)
               + bn1_ref[...])
    hn2 = jnp.dot(hn, wn2_ref[...],
                  preferred_element_type=jnp.float32) + bn2_ref[...]
    hN = _silu(hn2) + h
    hnew_ref[...] = hN
    xagg = xagg2_ref[0] + xagg2_ref[1]
    deg = xagg[:, 3:4]
    lane = lax.broadcasted_iota(jnp.int32, (1, XW), 1)
    xpnew_ref[...] = xp_ref[...] + jnp.where(lane < 3, xagg / (deg + 1.0), 0.0)
    pt_ref[...] = jnp.dot(hN, wea_ref[...],
                          preferred_element_type=jnp.float32) + be1n_ref[...]
    qt_ref[...] = jnp.dot(hN, web_ref[...], preferred_element_type=jnp.float32)


def _node_stage(h, agg2, xp, xagg2, wn1a, wn1b, bn1, wn2, bn2, wea, web, be1n):
    grid = (N // NODE_BLK,)
    bn = lambda w: pl.BlockSpec((NODE_BLK, w), lambda i: (i, 0))
    b2 = lambda w: pl.BlockSpec((NC, NODE_BLK, w), lambda i: (0, i, 0))
    full = lambda a, b: pl.BlockSpec((a, b), lambda i: (0, 0))
    return pl.pallas_call(
        _node_kernel,
        grid=grid,
        in_specs=[bn(H), b2(HW), bn(XW), b2(XW),
                  full(H, H), full(H, H), full(1, H), full(H, H), full(1, H),
                  full(H, H), full(H, H), full(1, H)],
        out_specs=[bn(H), bn(XW), bn(H), bn(H)],
        out_shape=[
            jax.ShapeDtypeStruct((N, H), jnp.float32),
            jax.ShapeDtypeStruct((N, XW), jnp.float32),
            jax.ShapeDtypeStruct((N, H), jnp.float32),
            jax.ShapeDtypeStruct((N, H), jnp.float32),
        ],
    )(h, agg2, xp, xagg2, wn1a, wn1b, bn1.reshape(1, H), wn2,
      bn2.reshape(1, H), wea, web, be1n.reshape(1, H))


def _nodef_kernel(h_ref, agg2_ref, xp_ref, xagg2_ref,
                  wn1a_ref, wn1b_ref, bn1_ref, wn2_ref, bn2_ref,
                  wout_ref, bout_ref,
                  hout_ref, xpnew_ref):
    h = h_ref[...]
    agg = jnp.concatenate([agg2_ref[0], agg2_ref[1]], axis=-1)
    hn = _silu(jnp.dot(h, wn1a_ref[...], preferred_element_type=jnp.float32)
               + jnp.dot(agg, wn1b_ref[...], preferred_element_type=jnp.float32)
               + bn1_ref[...])
    hn2 = jnp.dot(hn, wn2_ref[...],
                  preferred_element_type=jnp.float32) + bn2_ref[...]
    hN = _silu(hn2) + h
    hout_ref[...] = jnp.dot(hN, wout_ref[...],
                            preferred_element_type=jnp.float32) + bout_ref[...]
    xagg = xagg2_ref[0] + xagg2_ref[1]
    deg = xagg[:, 3:4]
    lane = lax.broadcasted_iota(jnp.int32, (1, XW), 1)
    xpnew_ref[...] = xp_ref[...] + jnp.where(lane < 3, xagg / (deg + 1.0), 0.0)


def _nodef_stage(h, agg2, xp, xagg2, wn1a, wn1b, bn1, wn2, bn2, W_out, b_out):
    grid = (N // NODE_BLK,)
    OUT = W_out.shape[1]
    bn = lambda w: pl.BlockSpec((NODE_BLK, w), lambda i: (i, 0))
    b2 = lambda w: pl.BlockSpec((NC, NODE_BLK, w), lambda i: (0, i, 0))
    full = lambda a, b: pl.BlockSpec((a, b), lambda i: (0, 0))
    return pl.pallas_call(
        _nodef_kernel,
        grid=grid,
        in_specs=[bn(H), b2(HW), bn(XW), b2(XW),
                  full(H, H), full(H, H), full(1, H), full(H, H), full(1, H),
                  full(H, OUT), full(1, OUT)],
        out_specs=[bn(OUT), bn(XW)],
        out_shape=[
            jax.ShapeDtypeStruct((N, OUT), jnp.float32),
            jax.ShapeDtypeStruct((N, XW), jnp.float32),
        ],
    )(h, agg2, xp, xagg2, wn1a, wn1b, bn1.reshape(1, H), wn2,
      bn2.reshape(1, H), W_out, b_out.reshape(1, OUT))


def kernel(h, x, edge_index, W_in, b_in, W_out, b_out, We1, be1, We2, be2,
           Wn1, bn1, Wn2, bn2, Wx, bx):
    srcv = edge_index[0]
    dstv = edge_index[1]
    xp = jnp.concatenate([x, jnp.zeros((N, XW - 3), jnp.float32)], axis=1)
    zeros_h = jnp.zeros((NP, XW), jnp.float32)

    h, P, Q = _node0_stage(h, W_in, b_in, We1[0, :H], We1[0, H:2 * H], be1[0])

    for l in range(DEPTH):
        w512b = We1[l, 2 * H].reshape(1, H).astype(jnp.bfloat16).astype(jnp.float32)
        epre, r = _gather(P, Q, xp, srcv, dstv)
        m2, rc = _edge_stage(epre, r, w512b,
                             be2[l].reshape(1, H),
                             We2[l],
                             Wx[l], bx[l].reshape(1, 1))
        agg2, xagg2 = _scatter(m2, rc, dstv, zeros_h)
        if l < DEPTH - 1:
            h, xp, P, Q = _node_stage(
                h, agg2, xp, xagg2,
                Wn1[l, :H], Wn1[l, H:], bn1[l], Wn2[l], bn2[l],
                We1[l + 1, :H], We1[l + 1, H:2 * H], be1[l + 1])
        else:
            h, xp = _nodef_stage(
                h, agg2, xp, xagg2,
                Wn1[l, :H], Wn1[l, H:], bn1[l], Wn2[l], bn2[l],
                W_out, b_out)
    return (h, xp[:, :3])
